# baseline (speedup 1.0000x reference)
import functools

import jax
import jax.numpy as jnp
from jax import lax
from jax.experimental import pallas as pl
from jax.experimental.pallas import tpu as pltpu
from jax.experimental.pallas import tpu_sc as plsc

VOCAB = 100000
VHALF = VOCAB // 2
EMB = 64
HID = 128
N_THEME = 10
N_SENTI = 3
B = 1024
L = 200
BL = B * L

LH0 = 96
LH1 = L - LH0

NC = 2
NSUB = 16
NW = NC * NSUB
DIMS_PER_W = EMB // NW
CHR = 8
CH = CHR * B
UNROLL = 8

LB = 8


def _sc_gather_t(tok_half, table_i32, lh):
    mesh = plsc.VectorSubcoreMesh(core_axis_name="c", subcore_axis_name="s")
    n_lc = lh // CHR

    @functools.partial(
        pl.kernel,
        out_type=jax.ShapeDtypeStruct((EMB, lh * B), jnp.float32),
        mesh=mesh,
        scratch_types=[
            pltpu.VMEM((VHALF,), jnp.int32),
            pltpu.VMEM((CHR, B), jnp.int32),
            pltpu.VMEM((CHR, B), jnp.int32),
            pltpu.VMEM((CH,), jnp.float32),
            pltpu.VMEM((CH,), jnp.float32),
            pltpu.SemaphoreType.DMA,
            pltpu.SemaphoreType.DMA,
            pltpu.SemaphoreType.DMA,
            pltpu.SemaphoreType.DMA,
        ],
        compiler_params=pltpu.CompilerParams(needs_layout_passes=False),
    )
    def gather_kernel(tok_hbm, table_hbm, out_hbm, row_v,
                      idx_a, idx_b, out_a, out_b,
                      isem_a, isem_b, osem_a, osem_b):
        wid = lax.axis_index("s") * NC + lax.axis_index("c")
        idx_bufs = (idx_a, idx_b)
        out_bufs = (out_a, out_b)
        isems = (isem_a, isem_b)
        osems = (osem_a, osem_b)

        def tok_slice(c):
            lo = c * CHR if isinstance(c, int) else pl.multiple_of(c * CHR, CHR)
            return tok_hbm.at[pl.ds(lo, CHR), :]

        def out_slice(d, c):
            off = c * CH if isinstance(c, int) else pl.multiple_of(c * CH, CH)
            return out_hbm.at[d, pl.ds(off, CH)]

        def gather_chunk(idx_v, out_v):
            for r in range(CHR):
                @plsc.parallel_loop(0, B, 16, unroll=UNROLL)
                def _g(base):
                    tok = idx_v[r, pl.ds(base, 16)]
                    word = plsc.load_gather(row_v, [tok >> 1])
                    bits = jnp.where((tok & 1) == 0,
                                     word << 16, word & -65536)
                    out_v[pl.ds(r * B + base, 16)] = plsc.bitcast(
                        bits, jnp.float32)

        def do_chunk(dd, d, c, p):
            idx_v, out_v = idx_bufs[p], out_bufs[p]
            isem, osem = isems[p], osems[p]
            pltpu.make_async_copy(tok_slice(c), idx_v, isem).wait()
            def _drain():
                pltpu.make_async_copy(out_v, out_slice(d, c), osem).wait()
            if dd == 0 and not isinstance(c, int):
                pl.when(c > 1)(_drain)
            else:
                _drain()
            gather_chunk(idx_v, out_v)
            pltpu.async_copy(out_v, out_slice(d, c), osem)
            if isinstance(c, int):
                if c + 2 < n_lc:
                    pltpu.async_copy(tok_slice(c + 2), idx_v, isem)
            else:
                @pl.when(c + 2 < n_lc)
                def _prefetch():
                    pltpu.async_copy(tok_slice(c + 2), idx_v, isem)

        for dd in range(DIMS_PER_W):
            d = wid * DIMS_PER_W + dd
            pltpu.sync_copy(table_hbm.at[d], row_v)
            pltpu.async_copy(tok_slice(0), idx_a, isem_a)
            pltpu.async_copy(tok_slice(1), idx_b, isem_b)

            def chunk_pair(ci, carry):
                do_chunk(dd, d, ci * 2, 0)
                do_chunk(dd, d, ci * 2 + 1, 1)
                return carry

            lax.fori_loop(0, n_lc // 2, chunk_pair, 0)
            if n_lc % 2:
                do_chunk(dd, d, n_lc - 1, (n_lc - 1) % 2)

        d_last = wid * DIMS_PER_W + DIMS_PER_W - 1
        for p in range(2):
            lastc = max(c for c in range(n_lc) if c % 2 == p)
            pltpu.make_async_copy(
                out_bufs[p], out_slice(d_last, lastc), osems[p]).wait()

    return gather_kernel(tok_half, table_i32)


def _make_tc_body(half, is_last, l_off, nstep):
    def tc_body(*refs):
        if half == 0:
            (len_ref, emb_ref, mt_ref, ms_ref, wt_ref, kt_ref, ws_ref,
             ks_ref, ut_ref, bt_ref, us_ref, bs_ref,
             o1_ref, o2_ref, acc_t, acc_s) = refs
        else:
            (len_ref, emb_ref, mt_ref, ms_ref, wt_ref, kt_ref, ws_ref,
             ks_ref, ut_ref, bt_ref, us_ref, bs_ref, pt_ref, ps_ref,
             o1_ref, o2_ref, acc_t, acc_s) = refs
        i = pl.program_id(0)

        @pl.when(i == 0)
        def _init():
            if half == 0:
                acc_t[...] = jnp.zeros((HID, B), jnp.float32)
                acc_s[...] = jnp.zeros((HID, B), jnp.float32)
            else:
                acc_t[...] = pt_ref[...]
                acc_s[...] = ps_ref[...]

        lens = len_ref[...]
        embx = emb_ref[...].astype(jnp.bfloat16)
        zt_emb = jnp.dot(wt_ref[...].astype(jnp.bfloat16), embx,
                         preferred_element_type=jnp.float32)
        zs_emb = jnp.dot(ws_ref[...].astype(jnp.bfloat16), embx,
                         preferred_element_type=jnp.float32)
        kt = kt_ref[...].astype(jnp.bfloat16)
        ks = ks_ref[...].astype(jnp.bfloat16)
        st = None
        ss = None
        for u in range(LB):
            mtx = mt_ref[:, u, 0, :].astype(jnp.bfloat16)
            msx = ms_ref[:, u, 0, :].astype(jnp.bfloat16)
            zt = zt_emb[:, u * B:(u + 1) * B] + jnp.dot(
                kt, mtx, preferred_element_type=jnp.float32)
            zs = zs_emb[:, u * B:(u + 1) * B] + jnp.dot(
                ks, msx, preferred_element_type=jnp.float32)
            w = jnp.where(lens > l_off + i * LB + u, 1.0, 0.0)
            ct = jnp.tanh(zt) * w
            cs = jnp.tanh(zs) * w
            st = ct if st is None else st + ct
            ss = cs if ss is None else ss + cs
        at = acc_t[...] + st
        as_ = acc_s[...] + ss

        if not is_last:
            @pl.when(i == nstep - 1)
            def _emit():
                o1_ref[...] = at
                o2_ref[...] = as_
        else:
            @pl.when(i == nstep - 1)
            def _fin():
                denom = jnp.maximum(lens.astype(jnp.float32), 1.0)
                pt = at / denom
                ps = as_ / denom
                o1_ref[...] = (jnp.dot(ut_ref[...], pt,
                                       preferred_element_type=jnp.float32)
                               + bt_ref[...])
                o2_ref[...] = (jnp.dot(us_ref[...], ps,
                                       preferred_element_type=jnp.float32)
                               + bs_ref[...])

        @pl.when(i < nstep - 1)
        def _carry():
            acc_t[...] = at
            acc_s[...] = as_

    return tc_body


def _tc_forward(half, len2, emb_t, mt_t, ms_t, wt_t, kt_t, ws_t, ks_t,
                ut_t, bt2, us_t, bs2, partial=None):
    is_last = half == 1
    lh = LH1 if half else LH0
    l_off = LH0 if half else 0
    nstep = lh // LB
    step0 = l_off // LB
    full2 = lambda shape: pl.BlockSpec(shape, lambda i: (0, 0))
    in_specs = [
        full2((1, B)),
        pl.BlockSpec((EMB, LB * B), lambda i: (0, i)),
        pl.BlockSpec((N_THEME, LB, 1, B), lambda i: (0, step0 + i, 0, 0)),
        pl.BlockSpec((N_SENTI, LB, 1, B), lambda i: (0, step0 + i, 0, 0)),
        full2((HID, EMB)),
        full2((HID, N_THEME)),
        full2((HID, EMB)),
        full2((HID, N_SENTI)),
        full2((N_THEME, HID)),
        full2((N_THEME, 1)),
        full2((N_SENTI, HID)),
        full2((N_SENTI, 1)),
    ]
    args = [len2, emb_t, mt_t, ms_t, wt_t, kt_t, ws_t, ks_t,
            ut_t, bt2, us_t, bs2]
    if partial is not None:
        in_specs += [full2((HID, B)), full2((HID, B))]
        args += list(partial)
    if is_last:
        out_shape = (jax.ShapeDtypeStruct((N_THEME, B), jnp.float32),
                     jax.ShapeDtypeStruct((N_SENTI, B), jnp.float32))
        out_specs = (pl.BlockSpec((N_THEME, B), lambda i: (0, 0)),
                     pl.BlockSpec((N_SENTI, B), lambda i: (0, 0)))
    else:
        out_shape = (jax.ShapeDtypeStruct((HID, B), jnp.float32),
                     jax.ShapeDtypeStruct((HID, B), jnp.float32))
        out_specs = (pl.BlockSpec((HID, B), lambda i: (0, 0)),
                     pl.BlockSpec((HID, B), lambda i: (0, 0)))
    return pl.pallas_call(
        _make_tc_body(half, is_last, l_off, nstep),
        grid=(nstep,),
        in_specs=in_specs,
        out_specs=out_specs,
        out_shape=out_shape,
        scratch_shapes=[
            pltpu.VMEM((HID, B), jnp.float32),
            pltpu.VMEM((HID, B), jnp.float32),
        ],
    )(*args)


def kernel(token, token_len, mention_theme, mention_senti, emb_table,
           W_theme, K_theme, W_senti, K_senti,
           U_theme, b_theme, U_senti, b_senti):
    tok_t = token.T
    table_bf = emb_table.T.astype(jnp.bfloat16)
    table_i32 = lax.bitcast_convert_type(
        table_bf.reshape(EMB, VHALF, 2), jnp.int32)

    len2 = token_len.reshape(1, B)
    mt_t = mention_theme.transpose(2, 1, 0).reshape(N_THEME, L, 1, B)
    ms_t = mention_senti.transpose(2, 1, 0).reshape(N_SENTI, L, 1, B)
    wargs = (W_theme.T, K_theme.T, W_senti.T, K_senti.T,
             U_theme.T, b_theme.reshape(N_THEME, 1),
             U_senti.T, b_senti.reshape(N_SENTI, 1))

    emb0 = _sc_gather_t(tok_t[:LH0], table_i32, LH0)
    emb1 = _sc_gather_t(tok_t[LH0:], table_i32, LH1)
    pt, ps = _tc_forward(0, len2, emb0, mt_t, ms_t, *wargs)
    out_t, out_s = _tc_forward(1, len2, emb1, mt_t, ms_t, *wargs,
                               partial=(pt, ps))
    return (out_t.T, out_s.T)


# speedup vs baseline: 2.8209x; 2.8209x over previous
import functools

import jax
import jax.numpy as jnp
from jax import lax
from jax.experimental import pallas as pl
from jax.experimental.pallas import tpu as pltpu
from jax.experimental.pallas import tpu_sc as plsc

VOCAB = 100000
EMB = 64
HID = 128
N_THEME = 10
N_SENTI = 3
B = 1024
L = 200
BL = B * L

NC = 2
NSUB = 16
NW = NC * NSUB
DIMS_PER_W = EMB // NW
CH = 6400
N_CH = BL // CH
UNROLL = 8

LB = 8


def _sc_gather_t(tok_flat, table_t):
    mesh = plsc.VectorSubcoreMesh(core_axis_name="c", subcore_axis_name="s")

    @functools.partial(
        pl.kernel,
        out_type=jax.ShapeDtypeStruct((EMB, BL), jnp.float32),
        mesh=mesh,
        scratch_types=[
            pltpu.VMEM((VOCAB,), jnp.float32),
            pltpu.VMEM((CH,), jnp.int32),
            pltpu.VMEM((CH,), jnp.int32),
            pltpu.VMEM((CH,), jnp.float32),
            pltpu.VMEM((CH,), jnp.float32),
            pltpu.SemaphoreType.DMA,
            pltpu.SemaphoreType.DMA,
            pltpu.SemaphoreType.DMA,
            pltpu.SemaphoreType.DMA,
        ],
        compiler_params=pltpu.CompilerParams(needs_layout_passes=False),
    )
    def gather_kernel(tok_hbm, table_hbm, out_hbm, row_v,
                      idx_a, idx_b, out_a, out_b,
                      isem_a, isem_b, osem_a, osem_b):
        wid = lax.axis_index("s") * NC + lax.axis_index("c")
        idx_bufs = (idx_a, idx_b)
        out_bufs = (out_a, out_b)
        isems = (isem_a, isem_b)
        osems = (osem_a, osem_b)

        def gather_chunk(idx_v, out_v):
            @plsc.parallel_loop(0, CH, 16, unroll=UNROLL)
            def _g(base):
                sl = pl.ds(base, 16)
                out_v[sl] = plsc.load_gather(row_v, [idx_v[sl]])

        for dd in range(DIMS_PER_W):
            d = wid * DIMS_PER_W + dd
            pltpu.sync_copy(table_hbm.at[d], row_v)
            pltpu.async_copy(tok_hbm.at[pl.ds(0, CH)], idx_a, isem_a)
            pltpu.async_copy(tok_hbm.at[pl.ds(CH, CH)], idx_b, isem_b)

            def chunk_pair(ci, carry):
                for p in range(2):
                    c = ci * 2 + p
                    off = c * CH
                    idx_v, out_v = idx_bufs[p], out_bufs[p]
                    isem, osem = isems[p], osems[p]
                    pltpu.make_async_copy(
                        tok_hbm.at[pl.ds(off, CH)], idx_v, isem).wait()
                    @pl.when(jnp.logical_or(c >= 2, dd > 0))
                    def _drain():
                        pltpu.make_async_copy(
                            out_v, out_hbm.at[d, pl.ds(off, CH)], osem).wait()
                    gather_chunk(idx_v, out_v)
                    pltpu.async_copy(
                        out_v, out_hbm.at[d, pl.ds(off, CH)], osem)
                    @pl.when(c + 2 < N_CH)
                    def _prefetch():
                        noff = (c + 2) * CH
                        pltpu.async_copy(
                            tok_hbm.at[pl.ds(noff, CH)], idx_v, isem)
                return carry

            lax.fori_loop(0, N_CH // 2, chunk_pair, 0)

        d_last = wid * DIMS_PER_W + DIMS_PER_W - 1
        for p in range(2):
            off = (N_CH - 2 + p) * CH
            pltpu.make_async_copy(
                out_bufs[p], out_hbm.at[d_last, pl.ds(off, CH)],
                osems[p]).wait()

    return gather_kernel(tok_flat, table_t)


def _tc_body(len_ref, emb_ref, mt_ref, ms_ref, wt_ref, kt_ref, ws_ref,
             ks_ref, ut_ref, bt_ref, us_ref, bs_ref, out_t_ref, out_s_ref,
             acc_t, acc_s):
    i = pl.program_id(0)

    @pl.when(i == 0)
    def _init():
        acc_t[...] = jnp.zeros((HID, B), jnp.float32)
        acc_s[...] = jnp.zeros((HID, B), jnp.float32)

    lens = len_ref[...]
    zt_emb = jnp.dot(wt_ref[...], emb_ref[...],
                     preferred_element_type=jnp.float32)
    zs_emb = jnp.dot(ws_ref[...], emb_ref[...],
                     preferred_element_type=jnp.float32)
    at = acc_t[...]
    as_ = acc_s[...]
    for u in range(LB):
        mtx = mt_ref[:, u, 0, :]
        msx = ms_ref[:, u, 0, :]
        zt = zt_emb[:, u * B:(u + 1) * B] + jnp.dot(
            kt_ref[...], mtx, preferred_element_type=jnp.float32)
        zs = zs_emb[:, u * B:(u + 1) * B] + jnp.dot(
            ks_ref[...], msx, preferred_element_type=jnp.float32)
        w = jnp.where(lens > i * LB + u, 1.0, 0.0)
        at = at + jnp.tanh(zt) * w
        as_ = as_ + jnp.tanh(zs) * w
    acc_t[...] = at
    acc_s[...] = as_

    @pl.when(i == (L // LB) - 1)
    def _fin():
        denom = jnp.maximum(lens.astype(jnp.float32), 1.0)
        pt = at / denom
        ps = as_ / denom
        out_t_ref[...] = (jnp.dot(ut_ref[...], pt,
                                  preferred_element_type=jnp.float32)
                          + bt_ref[...])
        out_s_ref[...] = (jnp.dot(us_ref[...], ps,
                                  preferred_element_type=jnp.float32)
                          + bs_ref[...])


def _tc_forward(len2, emb_t, mt_t, ms_t, wt_t, kt_t, ws_t, ks_t,
                ut_t, bt2, us_t, bs2):
    full2 = lambda shape: pl.BlockSpec(shape, lambda i: (0, 0))
    return pl.pallas_call(
        _tc_body,
        grid=(L // LB,),
        in_specs=[
            full2((1, B)),
            pl.BlockSpec((EMB, LB * B), lambda i: (0, i)),
            pl.BlockSpec((N_THEME, LB, 1, B), lambda i: (0, i, 0, 0)),
            pl.BlockSpec((N_SENTI, LB, 1, B), lambda i: (0, i, 0, 0)),
            full2((HID, EMB)),
            full2((HID, N_THEME)),
            full2((HID, EMB)),
            full2((HID, N_SENTI)),
            full2((N_THEME, HID)),
            full2((N_THEME, 1)),
            full2((N_SENTI, HID)),
            full2((N_SENTI, 1)),
        ],
        out_specs=(
            pl.BlockSpec((N_THEME, B), lambda i: (0, 0)),
            pl.BlockSpec((N_SENTI, B), lambda i: (0, 0)),
        ),
        out_shape=(
            jax.ShapeDtypeStruct((N_THEME, B), jnp.float32),
            jax.ShapeDtypeStruct((N_SENTI, B), jnp.float32),
        ),
        scratch_shapes=[
            pltpu.VMEM((HID, B), jnp.float32),
            pltpu.VMEM((HID, B), jnp.float32),
        ],
    )(len2, emb_t, mt_t, ms_t, wt_t, kt_t, ws_t, ks_t, ut_t, bt2, us_t, bs2)


def kernel(token, token_len, mention_theme, mention_senti, emb_table,
           W_theme, K_theme, W_senti, K_senti,
           U_theme, b_theme, U_senti, b_senti):
    tok_flat = token.T.reshape(BL)
    table_t = emb_table.T
    emb_t = _sc_gather_t(tok_flat, table_t)

    len2 = token_len.reshape(1, B)
    mt_t = mention_theme.transpose(2, 1, 0).reshape(N_THEME, L, 1, B)
    ms_t = mention_senti.transpose(2, 1, 0).reshape(N_SENTI, L, 1, B)
    out_t, out_s = _tc_forward(
        len2, emb_t, mt_t, ms_t,
        W_theme.T, K_theme.T, W_senti.T, K_senti.T,
        U_theme.T, b_theme.reshape(N_THEME, 1),
        U_senti.T, b_senti.reshape(N_SENTI, 1))
    return (out_t.T, out_s.T)
